# Initial kernel scaffold; baseline (speedup 1.0000x reference)
import jax
import jax.numpy as jnp
from jax.experimental import pallas as pl


def kernel(x, edge_index):
    raise NotImplementedError("write your pallas kernel here")


# speedup vs baseline: 9.5641x; 9.5641x over previous
import functools

import jax
import jax.numpy as jnp
from jax import lax
from jax.experimental import pallas as pl
from jax.experimental.pallas import tpu as pltpu
from jax.experimental.pallas import tpu_sc as plsc

N = 10000
E = 320000
D = 128
NC = 2
NS = 16
NW = NC * NS
R = 10240
RPT = R // NS
EPT = 10240
EPAD = NW * EPT
K = 128
NCHUNK = EPT // K
SLAB = 8
NSLAB = NCHUNK // SLAB

_mesh = plsc.VectorSubcoreMesh(core_axis_name="c", subcore_axis_name="s")


@functools.partial(
    pl.kernel,
    out_type=jax.ShapeDtypeStruct((NC, R), jnp.float32),
    mesh=_mesh,
    scratch_types=[
        pltpu.VMEM((NCHUNK, K), jnp.int32),
        pltpu.VMEM((K,), jnp.float32),
        pltpu.VMEM((RPT,), jnp.float32),
        pltpu.VMEM_SHARED((R,), jnp.float32),
    ],
)
def _degree_pass(dst_hbm, out_hbm, idx_v, ones_v, zrow_v, acc_sh):
    c = lax.axis_index("c")
    s = lax.axis_index("s")
    wid = c * NS + s

    pltpu.sync_copy(dst_hbm.at[wid], idx_v)
    for i in range(K // 16):
        ones_v[pl.ds(i * 16, 16)] = jnp.ones((16,), jnp.float32)
    for i in range(RPT // 16):
        zrow_v[pl.ds(i * 16, 16)] = jnp.zeros((16,), jnp.float32)
    pltpu.sync_copy(zrow_v, acc_sh.at[pl.ds(s * RPT, RPT)])
    plsc.subcore_barrier()

    def body(j, _):
        pltpu.sync_copy(ones_v, acc_sh.at[idx_v.at[j]], add=True)
        return _

    lax.fori_loop(0, NCHUNK, body, None)
    plsc.subcore_barrier()
    pltpu.sync_copy(acc_sh.at[pl.ds(s * RPT, RPT)],
                    out_hbm.at[c, pl.ds(s * RPT, RPT)])


@functools.partial(
    pl.kernel,
    out_type=jax.ShapeDtypeStruct((NC, R, D), jnp.float32),
    mesh=_mesh,
    scratch_types=[
        pltpu.VMEM((SLAB, K), jnp.int32),
        pltpu.VMEM((SLAB, K), jnp.int32),
        pltpu.VMEM((K, D), jnp.float32),
        pltpu.VMEM((K, D), jnp.float32),
        pltpu.VMEM_SHARED((R, D), jnp.float32),
        pltpu.SemaphoreType.DMA,
        pltpu.SemaphoreType.DMA,
    ],
)
def _message_pass(src_hbm, dst_hbm, norm_hbm, out_hbm,
                  src_v, dst_v, rows0, rows1, acc_sh, sem0, sem1):
    c = lax.axis_index("c")
    s = lax.axis_index("s")
    wid = c * NS + s

    def zero_body(i, _):
        for j in range(D // 16):
            rows0[i, pl.ds(j * 16, 16)] = jnp.zeros((16,), jnp.float32)
        return _

    lax.fori_loop(0, K, zero_body, None)
    for t in range(RPT // K):
        pltpu.sync_copy(rows0, acc_sh.at[pl.ds(s * RPT + t * K, K)])
    plsc.subcore_barrier()

    def slab_body(b, _):
        pltpu.sync_copy(src_hbm.at[wid, pl.ds(b * SLAB, SLAB)], src_v)
        pltpu.sync_copy(dst_hbm.at[wid, pl.ds(b * SLAB, SLAB)], dst_v)

        def body(t, _):
            j0 = 2 * t
            j1 = 2 * t + 1
            g0 = pltpu.async_copy(norm_hbm.at[src_v.at[j0]], rows0, sem0)
            g1 = pltpu.async_copy(norm_hbm.at[src_v.at[j1]], rows1, sem1)
            g0.wait()
            pltpu.sync_copy(rows0, acc_sh.at[dst_v.at[j0]], add=True)
            g1.wait()
            pltpu.sync_copy(rows1, acc_sh.at[dst_v.at[j1]], add=True)
            return _

        lax.fori_loop(0, SLAB // 2, body, None)
        return _

    lax.fori_loop(0, NSLAB, slab_body, None)
    plsc.subcore_barrier()
    pltpu.sync_copy(acc_sh.at[pl.ds(s * RPT, RPT)],
                    out_hbm.at[c, pl.ds(s * RPT, RPT)])


def _normalize_body(deg_ref, x_ref, out_ref):
    dsum = deg_ref[0] + deg_ref[1] + 1.0
    isq = lax.rsqrt(dsum)
    out_ref[...] = isq[:N] * x_ref[...]


_normalize = pl.pallas_call(
    _normalize_body,
    out_shape=jax.ShapeDtypeStruct((N, D), jnp.float32),
)


def _combine_body(deg_ref, part_ref, nrm_ref, out_ref):
    dsum = deg_ref[0] + deg_ref[1] + 1.0
    isq = lax.rsqrt(dsum)
    pooled = part_ref[0][:N] + part_ref[1][:N]
    out_ref[...] = isq[:N] * (pooled + nrm_ref[...])


_combine = pl.pallas_call(
    _combine_body,
    out_shape=jax.ShapeDtypeStruct((N, D), jnp.float32),
)


@jax.jit
def kernel(x, edge_index):
    src = edge_index[0].astype(jnp.int32)
    dst = edge_index[1].astype(jnp.int32)
    npad = EPAD - E
    pad_src = (jnp.arange(npad, dtype=jnp.int32) * 13) % N
    pad_dst = N + (jnp.arange(npad, dtype=jnp.int32) % (R - N))
    src3 = jnp.concatenate([src, pad_src]).reshape(NW, NCHUNK, K)
    dst3 = jnp.concatenate([dst, pad_dst]).reshape(NW, NCHUNK, K)

    degp = _degree_pass(dst3)
    degp3 = degp.reshape(NC, R, 1)
    normalized = _normalize(degp3, x)
    parts = _message_pass(src3, dst3, normalized)
    return _combine(degp3, parts, normalized)


# speedup vs baseline: 10.0925x; 1.0552x over previous
import functools

import jax
import jax.numpy as jnp
from jax import lax
from jax.experimental import pallas as pl
from jax.experimental.pallas import tpu as pltpu
from jax.experimental.pallas import tpu_sc as plsc

N = 10000
E = 320000
D = 128
NC = 2
NS = 16
NW = NC * NS
R = 10240
RPT = R // NS
EPT = 10240
EPAD = NW * EPT
K = 128
SLABC = 8
NSLAB = 10
NCHUNK = EPT // K

_mesh = plsc.VectorSubcoreMesh(core_axis_name="c", subcore_axis_name="s")


@functools.partial(
    pl.kernel,
    out_type=jax.ShapeDtypeStruct((NC, 1, R), jnp.float32),
    mesh=_mesh,
    scratch_types=[
        pltpu.VMEM((NCHUNK, K), jnp.int32),
        pltpu.VMEM((K,), jnp.float32),
        pltpu.VMEM((640,), jnp.float32),
        pltpu.VMEM_SHARED((R,), jnp.float32),
    ],
)
def _degree_pass(dst_hbm, out_hbm, idx_v, ones_v, zrow_v, acc_sh):
    c = lax.axis_index("c")
    s = lax.axis_index("s")
    wid = c * NS + s

    pltpu.sync_copy(dst_hbm.at[wid], idx_v)
    for i in range(K // 16):
        ones_v[pl.ds(i * 16, 16)] = jnp.ones((16,), jnp.float32)
    for i in range(640 // 16):
        zrow_v[pl.ds(i * 16, 16)] = jnp.zeros((16,), jnp.float32)
    pltpu.sync_copy(zrow_v.at[pl.ds(0, RPT)], acc_sh.at[pl.ds(s * RPT, RPT)])
    plsc.subcore_barrier()

    def body(j, _):
        pltpu.sync_copy(ones_v, acc_sh.at[idx_v.at[j]], add=True)
        return _

    lax.fori_loop(0, NCHUNK, body, None)
    plsc.subcore_barrier()

    @pl.when(s == 0)
    def _dump():
        pltpu.sync_copy(acc_sh, out_hbm.at[c, 0])


@functools.partial(
    pl.kernel,
    out_type=jax.ShapeDtypeStruct((NC, R, D), jnp.float32),
    mesh=_mesh,
    scratch_types=[
        pltpu.VMEM((SLABC, K), jnp.int32),
        pltpu.VMEM((SLABC, K), jnp.int32),
        pltpu.VMEM((SLABC, K), jnp.int32),
        pltpu.VMEM((SLABC, K), jnp.int32),
        pltpu.VMEM((K, D), jnp.float32),
        pltpu.VMEM((K, D), jnp.float32),
        pltpu.VMEM_SHARED((R, D), jnp.float32),
        pltpu.SemaphoreType.DMA,
        pltpu.SemaphoreType.DMA,
        pltpu.SemaphoreType.DMA,
        pltpu.SemaphoreType.DMA,
        pltpu.SemaphoreType.DMA,
        pltpu.SemaphoreType.DMA,
    ],
)
def _message_pass(src_hbm, dst_hbm, norm_hbm, out_hbm,
                  srcA, dstA, srcB, dstB, rows0, rows1, acc_sh,
                  siA, siB, sg0, sg1, ss0, ss1):
    c = lax.axis_index("c")
    s = lax.axis_index("s")
    wid = c * NS + s

    def _load_slab(b, sbuf, dbuf, sem):
        pltpu.async_copy(src_hbm.at[wid, pl.ds(b * SLABC, SLABC)], sbuf, sem)
        pltpu.async_copy(dst_hbm.at[wid, pl.ds(b * SLABC, SLABC)], dbuf, sem)

    def _wait_slab(b, sbuf, dbuf, sem):
        pltpu.make_async_copy(
            src_hbm.at[wid, pl.ds(b * SLABC, SLABC)], sbuf, sem).wait()
        pltpu.make_async_copy(
            dst_hbm.at[wid, pl.ds(b * SLABC, SLABC)], dbuf, sem).wait()

    _load_slab(0, srcA, dstA, siA)

    def zero_body(i, _):
        for j in range(D // 16):
            rows0[i, pl.ds(j * 16, 16)] = jnp.zeros((16,), jnp.float32)
        return _

    lax.fori_loop(0, K, zero_body, None)
    for t in range(RPT // K):
        pltpu.sync_copy(rows0, acc_sh.at[pl.ds(s * RPT + t * K, K)])
    _wait_slab(0, srcA, dstA, siA)
    plsc.subcore_barrier()

    bufs = [(srcA, dstA, siA), (srcB, dstB, siB)]
    rbufs = [(rows0, sg0, ss0), (rows1, sg1, ss1)]
    pltpu.async_copy(norm_hbm.at[srcA.at[0]], rows0, sg0)
    pltpu.async_copy(norm_hbm.at[srcA.at[1]], rows1, sg1)

    for b in range(NSLAB):
        cs, cd, _ = bufs[b % 2]
        ns, nd, nsem = bufs[(b + 1) % 2]
        last_slab = b + 1 >= NSLAB
        if not last_slab:
            _load_slab(b + 1, ns, nd, nsem)
        for t in range(SLABC // 2):
            for h in range(2):
                ch = 2 * t + h
                rows, sg, ss = rbufs[h]
                pltpu.make_async_copy(
                    norm_hbm.at[cs.at[ch]], rows, sg).wait()
                pltpu.async_copy(rows, acc_sh.at[cd.at[ch]], ss, add=True)
            if t == SLABC // 2 - 1 and not last_slab:
                _wait_slab(b + 1, ns, nd, nsem)
            for h in range(2):
                ch = 2 * t + h
                rows, sg, ss = rbufs[h]
                if ch + 2 < SLABC:
                    psrc, pc = cs, ch + 2
                elif last_slab:
                    psrc, pc = cs, ch
                else:
                    psrc, pc = ns, h
                pltpu.make_async_copy(rows, acc_sh.at[cd.at[ch]], ss).wait()
                pltpu.async_copy(norm_hbm.at[psrc.at[pc]], rows, sg)

    pltpu.make_async_copy(norm_hbm.at[srcA.at[0]], rows0, sg0).wait()
    pltpu.make_async_copy(norm_hbm.at[srcA.at[1]], rows1, sg1).wait()
    plsc.subcore_barrier()
    pltpu.sync_copy(acc_sh.at[pl.ds(s * RPT, RPT)],
                    out_hbm.at[c, pl.ds(s * RPT, RPT)])


def _normalize_body(deg_ref, x_ref, out_ref):
    dsum = deg_ref[0] + deg_ref[1] + 1.0
    isq = lax.rsqrt(dsum)
    out_ref[...] = isq[:N] * x_ref[...]


_normalize = pl.pallas_call(
    _normalize_body,
    out_shape=jax.ShapeDtypeStruct((N, D), jnp.float32),
)


def _combine_body(deg_ref, part_ref, nrm_ref, out_ref):
    dsum = deg_ref[0] + deg_ref[1] + 1.0
    isq = lax.rsqrt(dsum)
    pooled = part_ref[0][:N] + part_ref[1][:N]
    out_ref[...] = isq[:N] * (pooled + nrm_ref[...])


_combine = pl.pallas_call(
    _combine_body,
    out_shape=jax.ShapeDtypeStruct((N, D), jnp.float32),
)


@jax.jit
def kernel(x, edge_index):
    src = edge_index[0].astype(jnp.int32)
    dst = edge_index[1].astype(jnp.int32)
    npad = EPAD - E
    pad_src = (jnp.arange(npad, dtype=jnp.int32) * 13) % N
    pad_dst = N + (jnp.arange(npad, dtype=jnp.int32) % (R - N))
    src3 = jnp.concatenate([src, pad_src]).reshape(NW, NCHUNK, K)
    dst3 = jnp.concatenate([dst, pad_dst]).reshape(NW, NCHUNK, K)

    degp = _degree_pass(dst3)
    degp3 = degp.reshape(NC, R, 1)
    normalized = _normalize(degp3, x)
    parts = _message_pass(src3, dst3, normalized)
    return _combine(degp3, parts, normalized)


# speedup vs baseline: 11.2755x; 1.1172x over previous
import functools

import jax
import jax.numpy as jnp
from jax import lax
from jax.experimental import pallas as pl
from jax.experimental.pallas import tpu as pltpu
from jax.experimental.pallas import tpu_sc as plsc

N = 10000
E = 320000
D = 128
NC = 2
NS = 16
NW = NC * NS
R = 10240
RPT = R // NS
EPT = 10240
EPAD = NW * EPT
K = 128
MK = 64
MCHUNK = EPT // MK
SLABC = 8
NSLAB = MCHUNK // SLABC
NCHUNK = EPT // K
NRB = 4

_mesh = plsc.VectorSubcoreMesh(core_axis_name="c", subcore_axis_name="s")


@functools.partial(
    pl.kernel,
    out_type=jax.ShapeDtypeStruct((NC, 1, R), jnp.float32),
    mesh=_mesh,
    scratch_types=[
        pltpu.VMEM((NCHUNK, K), jnp.int32),
        pltpu.VMEM((K,), jnp.float32),
        pltpu.VMEM((640,), jnp.float32),
        pltpu.VMEM_SHARED((R,), jnp.float32),
    ],
)
def _degree_pass(dst_hbm, out_hbm, idx_v, ones_v, zrow_v, acc_sh):
    c = lax.axis_index("c")
    s = lax.axis_index("s")
    wid = c * NS + s

    pltpu.sync_copy(dst_hbm.at[wid], idx_v)
    for i in range(K // 16):
        ones_v[pl.ds(i * 16, 16)] = jnp.ones((16,), jnp.float32)
    for i in range(640 // 16):
        zrow_v[pl.ds(i * 16, 16)] = jnp.zeros((16,), jnp.float32)
    pltpu.sync_copy(zrow_v.at[pl.ds(0, RPT)], acc_sh.at[pl.ds(s * RPT, RPT)])
    plsc.subcore_barrier()

    def body(j, _):
        pltpu.sync_copy(ones_v, acc_sh.at[idx_v.at[j]], add=True)
        return _

    lax.fori_loop(0, NCHUNK, body, None)
    plsc.subcore_barrier()

    @pl.when(s == 0)
    def _dump():
        pltpu.sync_copy(acc_sh, out_hbm.at[c, 0])


@functools.partial(
    pl.kernel,
    out_type=jax.ShapeDtypeStruct((NC, R, D), jnp.float32),
    mesh=_mesh,
    scratch_types=(
        [pltpu.VMEM((SLABC, MK), jnp.int32)] * 4
        + [pltpu.VMEM((MK, D), jnp.float32)] * NRB
        + [pltpu.VMEM_SHARED((R, D), jnp.float32)]
        + [pltpu.SemaphoreType.DMA] * (2 + 2 * NRB)
    ),
)
def _message_pass(src_hbm, dst_hbm, norm_hbm, out_hbm,
                  srcA, dstA, srcB, dstB, r0, r1, r2, r3, acc_sh,
                  siA, siB, sg0, sg1, sg2, sg3, ss0, ss1, ss2, ss3):
    c = lax.axis_index("c")
    s = lax.axis_index("s")
    wid = c * NS + s
    idx = [(srcA, dstA, siA), (srcB, dstB, siB)]
    ring = [(r0, sg0, ss0), (r1, sg1, ss1), (r2, sg2, ss2), (r3, sg3, ss3)]

    def _load_slab(b):
        sbuf, dbuf, sem = idx[b % 2]
        pltpu.async_copy(src_hbm.at[wid, pl.ds(b * SLABC, SLABC)], sbuf, sem)
        pltpu.async_copy(dst_hbm.at[wid, pl.ds(b * SLABC, SLABC)], dbuf, sem)

    def _wait_slab(b):
        sbuf, dbuf, sem = idx[b % 2]
        pltpu.make_async_copy(
            src_hbm.at[wid, pl.ds(b * SLABC, SLABC)], sbuf, sem).wait()
        pltpu.make_async_copy(
            dst_hbm.at[wid, pl.ds(b * SLABC, SLABC)], dbuf, sem).wait()

    def _gather(ch, rows, sem):
        sbuf = idx[(ch // SLABC) % 2][0]
        pltpu.async_copy(norm_hbm.at[sbuf.at[ch % SLABC]], rows, sem)

    def _wait_gather(ch, rows, sem):
        sbuf = idx[(ch // SLABC) % 2][0]
        pltpu.make_async_copy(
            norm_hbm.at[sbuf.at[ch % SLABC]], rows, sem).wait()

    def _scatter(ch, rows, sem):
        dbuf = idx[(ch // SLABC) % 2][1]
        pltpu.async_copy(rows, acc_sh.at[dbuf.at[ch % SLABC]], sem, add=True)

    def _wait_scatter(ch, rows, sem):
        dbuf = idx[(ch // SLABC) % 2][1]
        pltpu.make_async_copy(rows, acc_sh.at[dbuf.at[ch % SLABC]], sem).wait()

    _load_slab(0)

    def zero_body(i, _):
        for j in range(D // 16):
            r0[i, pl.ds(j * 16, 16)] = jnp.zeros((16,), jnp.float32)
        return _

    lax.fori_loop(0, MK, zero_body, None)
    for t in range(RPT // MK):
        pltpu.sync_copy(r0, acc_sh.at[pl.ds(s * RPT + t * MK, MK)])
    _wait_slab(0)
    plsc.subcore_barrier()

    _gather(0, r0, sg0)
    _gather(1, r1, sg1)
    for ch in range(MCHUNK):
        b = ch // SLABC
        cin = ch % SLABC
        if cin == 0 and b + 1 <= NSLAB - 1:
            _load_slab(b + 1)
        if cin == SLABC - 2 and b + 1 <= NSLAB - 1:
            _wait_slab(b + 1)
        rows, sg, ss = ring[ch % NRB]
        _wait_gather(ch, rows, sg)
        _scatter(ch, rows, ss)
        prows, psg, pss = ring[(ch + 2) % NRB]
        if ch >= 2:
            _wait_scatter(ch - 2, prows, pss)
        pch = ch + 2 if ch + 2 < MCHUNK else ch - 2
        _gather(pch, prows, psg)

    for ch in (MCHUNK - 2, MCHUNK - 1):
        rows, sg, ss = ring[ch % NRB]
        _wait_scatter(ch, rows, ss)
    for ch in (MCHUNK - 4, MCHUNK - 3):
        rows, sg, ss = ring[ch % NRB]
        _wait_gather(ch, rows, sg)
    plsc.subcore_barrier()
    pltpu.sync_copy(acc_sh.at[pl.ds(s * RPT, RPT)],
                    out_hbm.at[c, pl.ds(s * RPT, RPT)])


def _normalize_body(deg_ref, x_ref, out_ref):
    dsum = deg_ref[0] + deg_ref[1] + 1.0
    isq = lax.rsqrt(dsum)
    out_ref[...] = isq[:N] * x_ref[...]


_normalize = pl.pallas_call(
    _normalize_body,
    out_shape=jax.ShapeDtypeStruct((N, D), jnp.float32),
)


def _combine_body(deg_ref, part_ref, nrm_ref, out_ref):
    dsum = deg_ref[0] + deg_ref[1] + 1.0
    isq = lax.rsqrt(dsum)
    pooled = part_ref[0][:N] + part_ref[1][:N]
    out_ref[...] = isq[:N] * (pooled + nrm_ref[...])


_combine = pl.pallas_call(
    _combine_body,
    out_shape=jax.ShapeDtypeStruct((N, D), jnp.float32),
)


@jax.jit
def kernel(x, edge_index):
    src = edge_index[0].astype(jnp.int32)
    dst = edge_index[1].astype(jnp.int32)
    npad = EPAD - E
    pad_src = (jnp.arange(npad, dtype=jnp.int32) * 13) % N
    pad_dst = N + (jnp.arange(npad, dtype=jnp.int32) % (R - N))
    src_p = jnp.concatenate([src, pad_src])
    dst_p = jnp.concatenate([dst, pad_dst])

    degp = _degree_pass(dst_p.reshape(NW, NCHUNK, K))
    degp3 = degp.reshape(NC, R, 1)
    normalized = _normalize(degp3, x)
    parts = _message_pass(src_p.reshape(NW, MCHUNK, MK),
                          dst_p.reshape(NW, MCHUNK, MK),
                          normalized)
    return _combine(degp3, parts, normalized)


# speedup vs baseline: 11.5407x; 1.0235x over previous
import functools

import jax
import jax.numpy as jnp
from jax import lax
from jax.experimental import pallas as pl
from jax.experimental.pallas import tpu as pltpu
from jax.experimental.pallas import tpu_sc as plsc

N = 10000
E = 320000
D = 128
NC = 2
NS = 16
NW = NC * NS
R = 10240
RPT = R // NS
EPT = 10240
EPAD = NW * EPT
K = 128
MK = 64
MCHUNK = EPT // MK
SLABC = 8
NSLAB = MCHUNK // SLABC
NCHUNK = EPT // K
NRB = 4

_mesh = plsc.VectorSubcoreMesh(core_axis_name="c", subcore_axis_name="s")


@functools.partial(
    pl.kernel,
    out_type=jax.ShapeDtypeStruct((NC, 1, R), jnp.float32),
    mesh=_mesh,
    scratch_types=[
        pltpu.VMEM((MCHUNK, MK), jnp.int32),
        pltpu.VMEM((MK,), jnp.float32),
        pltpu.VMEM((640,), jnp.float32),
        pltpu.VMEM_SHARED((R,), jnp.float32),
    ],
)
def _degree_pass(dst_hbm, out_hbm, idx_v, ones_v, zrow_v, acc_sh):
    c = lax.axis_index("c")
    s = lax.axis_index("s")
    wid = c * NS + s

    pltpu.sync_copy(dst_hbm.at[wid], idx_v)
    for i in range(MK // 16):
        ones_v[pl.ds(i * 16, 16)] = jnp.ones((16,), jnp.float32)
    for i in range(640 // 16):
        zrow_v[pl.ds(i * 16, 16)] = jnp.zeros((16,), jnp.float32)
    pltpu.sync_copy(zrow_v.at[pl.ds(0, RPT)], acc_sh.at[pl.ds(s * RPT, RPT)])
    plsc.subcore_barrier()

    def body(j, _):
        pltpu.sync_copy(ones_v, acc_sh.at[idx_v.at[j]], add=True)
        return _

    lax.fori_loop(0, MCHUNK, body, None)
    plsc.subcore_barrier()

    @pl.when(s == 0)
    def _dump():
        pltpu.sync_copy(acc_sh, out_hbm.at[c, 0])


@functools.partial(
    pl.kernel,
    out_type=jax.ShapeDtypeStruct((NC, R, D), jnp.float32),
    mesh=_mesh,
    scratch_types=(
        [pltpu.VMEM((SLABC, MK), jnp.int32)] * 4
        + [pltpu.VMEM((MK, D), jnp.float32)] * NRB
        + [pltpu.VMEM_SHARED((R, D), jnp.float32)]
        + [pltpu.SemaphoreType.DMA] * (2 + 2 * NRB)
    ),
)
def _message_pass(src_hbm, dst_hbm, norm_hbm, out_hbm,
                  srcA, dstA, srcB, dstB, r0, r1, r2, r3, acc_sh,
                  siA, siB, sg0, sg1, sg2, sg3, ss0, ss1, ss2, ss3):
    c = lax.axis_index("c")
    s = lax.axis_index("s")
    wid = c * NS + s
    idx = [(srcA, dstA, siA), (srcB, dstB, siB)]
    ring = [(r0, sg0, ss0), (r1, sg1, ss1), (r2, sg2, ss2), (r3, sg3, ss3)]

    def _load_slab(b):
        sbuf, dbuf, sem = idx[b % 2]
        pltpu.async_copy(src_hbm.at[wid, pl.ds(b * SLABC, SLABC)], sbuf, sem)
        pltpu.async_copy(dst_hbm.at[wid, pl.ds(b * SLABC, SLABC)], dbuf, sem)

    def _wait_slab(b):
        sbuf, dbuf, sem = idx[b % 2]
        pltpu.make_async_copy(
            src_hbm.at[wid, pl.ds(b * SLABC, SLABC)], sbuf, sem).wait()
        pltpu.make_async_copy(
            dst_hbm.at[wid, pl.ds(b * SLABC, SLABC)], dbuf, sem).wait()

    def _gather(ch, rows, sem):
        sbuf = idx[(ch // SLABC) % 2][0]
        pltpu.async_copy(norm_hbm.at[sbuf.at[ch % SLABC]], rows, sem)

    def _wait_gather(ch, rows, sem):
        sbuf = idx[(ch // SLABC) % 2][0]
        pltpu.make_async_copy(
            norm_hbm.at[sbuf.at[ch % SLABC]], rows, sem).wait()

    def _scatter(ch, rows, sem):
        dbuf = idx[(ch // SLABC) % 2][1]
        pltpu.async_copy(rows, acc_sh.at[dbuf.at[ch % SLABC]], sem, add=True)

    def _wait_scatter(ch, rows, sem):
        dbuf = idx[(ch // SLABC) % 2][1]
        pltpu.make_async_copy(rows, acc_sh.at[dbuf.at[ch % SLABC]], sem).wait()

    _load_slab(0)

    def zero_body(i, _):
        for j in range(D // 16):
            r0[i, pl.ds(j * 16, 16)] = jnp.zeros((16,), jnp.float32)
        return _

    lax.fori_loop(0, MK, zero_body, None)
    for t in range(RPT // MK):
        pltpu.sync_copy(r0, acc_sh.at[pl.ds(s * RPT + t * MK, MK)])
    _wait_slab(0)
    plsc.subcore_barrier()

    _gather(0, r0, sg0)
    _gather(1, r1, sg1)
    for ch in range(MCHUNK):
        b = ch // SLABC
        cin = ch % SLABC
        if cin == 0 and b + 1 <= NSLAB - 1:
            _load_slab(b + 1)
        if cin == SLABC - 2 and b + 1 <= NSLAB - 1:
            _wait_slab(b + 1)
        rows, sg, ss = ring[ch % NRB]
        _wait_gather(ch, rows, sg)
        _scatter(ch, rows, ss)
        prows, psg, pss = ring[(ch + 2) % NRB]
        if ch >= 2:
            _wait_scatter(ch - 2, prows, pss)
        pch = ch + 2 if ch + 2 < MCHUNK else ch - 2
        _gather(pch, prows, psg)

    for ch in (MCHUNK - 2, MCHUNK - 1):
        rows, sg, ss = ring[ch % NRB]
        _wait_scatter(ch, rows, ss)
    for ch in (MCHUNK - 4, MCHUNK - 3):
        rows, sg, ss = ring[ch % NRB]
        _wait_gather(ch, rows, sg)
    plsc.subcore_barrier()
    pltpu.sync_copy(acc_sh.at[pl.ds(s * RPT, RPT)],
                    out_hbm.at[c, pl.ds(s * RPT, RPT)])


def _row_scale(deg_ref):
    dsum = deg_ref[0] + deg_ref[1] + 1.0
    isq = lax.rsqrt(dsum)
    ones = jnp.ones((1, D), jnp.float32)
    return lax.dot_general(isq, ones, (((0,), (0,)), ((), ())),
                           precision=lax.Precision.HIGHEST,
                           preferred_element_type=jnp.float32)


def _normalize_body(deg_ref, x_ref, out_ref):
    omat = _row_scale(deg_ref)
    out_ref[...] = omat[:N] * x_ref[...]


_normalize = pl.pallas_call(
    _normalize_body,
    out_shape=jax.ShapeDtypeStruct((N, D), jnp.float32),
)


def _combine_body(deg_ref, part_ref, nrm_ref, out_ref):
    omat = _row_scale(deg_ref)
    pooled = part_ref[0][:N] + part_ref[1][:N]
    out_ref[...] = omat[:N] * (pooled + nrm_ref[...])


_combine = pl.pallas_call(
    _combine_body,
    out_shape=jax.ShapeDtypeStruct((N, D), jnp.float32),
)


@jax.jit
def kernel(x, edge_index):
    src = edge_index[0].astype(jnp.int32)
    dst = edge_index[1].astype(jnp.int32)
    npad = EPAD - E
    pad = jnp.arange(npad, dtype=jnp.int32)
    pad_src = pad & 8191
    pad_dst = N + (pad & 127)
    src3 = jnp.concatenate([src, pad_src]).reshape(NW, MCHUNK, MK)
    dst3 = jnp.concatenate([dst, pad_dst]).reshape(NW, MCHUNK, MK)

    degp = _degree_pass(dst3)
    normalized = _normalize(degp, x)
    parts = _message_pass(src3, dst3, normalized)
    return _combine(degp, parts, normalized)


# speedup vs baseline: 12.0806x; 1.0468x over previous
import functools

import jax
import jax.numpy as jnp
from jax import lax
from jax.experimental import pallas as pl
from jax.experimental.pallas import tpu as pltpu
from jax.experimental.pallas import tpu_sc as plsc

N = 10000
E = 320000
D = 128
NC = 2
NS = 16
NW = NC * NS
R = 10240
RPT = R // NS
EPT = 10240
EPAD = NW * EPT
K = 128
MK = 64
MCHUNK = EPT // MK
SLABC = 8
NSLAB = MCHUNK // SLABC
NCHUNK = EPT // K
NRB = 4

_mesh = plsc.VectorSubcoreMesh(core_axis_name="c", subcore_axis_name="s")


@functools.partial(
    pl.kernel,
    out_type=jax.ShapeDtypeStruct((NC, 1, R), jnp.float32),
    mesh=_mesh,
    scratch_types=[
        pltpu.VMEM((MCHUNK, MK), jnp.int32),
        pltpu.VMEM((MK,), jnp.float32),
        pltpu.VMEM((640,), jnp.float32),
        pltpu.VMEM_SHARED((R,), jnp.float32),
        pltpu.SemaphoreType.DMA,
        pltpu.SemaphoreType.DMA,
    ],
)
def _degree_pass(dst_hbm, out_hbm, idx_v, ones_v, zrow_v, acc_sh, d0, d1):
    c = lax.axis_index("c")
    s = lax.axis_index("s")
    wid = c * NS + s
    dsem = (d0, d1)

    pltpu.sync_copy(dst_hbm.at[wid], idx_v)
    for i in range(MK // 16):
        ones_v[pl.ds(i * 16, 16)] = jnp.ones((16,), jnp.float32)
    for i in range(640 // 16):
        zrow_v[pl.ds(i * 16, 16)] = jnp.zeros((16,), jnp.float32)
    pltpu.sync_copy(zrow_v.at[pl.ds(0, RPT)], acc_sh.at[pl.ds(s * RPT, RPT)])
    plsc.subcore_barrier()

    GRP = 8
    NG = MCHUNK // GRP
    for g in range(NG):
        for t in range(GRP):
            pltpu.async_copy(ones_v, acc_sh.at[idx_v.at[g * GRP + t]],
                             dsem[g % 2], add=True)
        if g >= 1:
            for t in range(GRP):
                pltpu.make_async_copy(
                    ones_v, acc_sh.at[idx_v.at[(g - 1) * GRP + t]],
                    dsem[(g - 1) % 2]).wait()
    for t in range(GRP):
        pltpu.make_async_copy(
            ones_v, acc_sh.at[idx_v.at[(NG - 1) * GRP + t]],
            dsem[(NG - 1) % 2]).wait()
    plsc.subcore_barrier()

    @pl.when(s == 0)
    def _dump():
        pltpu.sync_copy(acc_sh, out_hbm.at[c, 0])


@functools.partial(
    pl.kernel,
    out_type=jax.ShapeDtypeStruct((NC, R, D), jnp.float32),
    mesh=_mesh,
    scratch_types=(
        [pltpu.VMEM((SLABC, MK), jnp.int32)] * 4
        + [pltpu.VMEM((MK, D), jnp.float32)] * NRB
        + [pltpu.VMEM_SHARED((R, D), jnp.float32)]
        + [pltpu.SemaphoreType.DMA] * (2 + 2 * NRB)
    ),
)
def _message_pass(src_hbm, dst_hbm, norm_hbm, out_hbm,
                  srcA, dstA, srcB, dstB, r0, r1, r2, r3, acc_sh,
                  siA, siB, sg0, sg1, sg2, sg3, ss0, ss1, ss2, ss3):
    c = lax.axis_index("c")
    s = lax.axis_index("s")
    wid = c * NS + s
    idx = [(srcA, dstA, siA), (srcB, dstB, siB)]
    ring = [(r0, sg0, ss0), (r1, sg1, ss1), (r2, sg2, ss2), (r3, sg3, ss3)]

    def _load_slab(b):
        sbuf, dbuf, sem = idx[b % 2]
        pltpu.async_copy(src_hbm.at[wid, pl.ds(b * SLABC, SLABC)], sbuf, sem)
        pltpu.async_copy(dst_hbm.at[wid, pl.ds(b * SLABC, SLABC)], dbuf, sem)

    def _wait_slab(b):
        sbuf, dbuf, sem = idx[b % 2]
        pltpu.make_async_copy(
            src_hbm.at[wid, pl.ds(b * SLABC, SLABC)], sbuf, sem).wait()
        pltpu.make_async_copy(
            dst_hbm.at[wid, pl.ds(b * SLABC, SLABC)], dbuf, sem).wait()

    def _gather(ch, rows, sem):
        sbuf = idx[(ch // SLABC) % 2][0]
        pltpu.async_copy(norm_hbm.at[sbuf.at[ch % SLABC]], rows, sem)

    def _wait_gather(ch, rows, sem):
        sbuf = idx[(ch // SLABC) % 2][0]
        pltpu.make_async_copy(
            norm_hbm.at[sbuf.at[ch % SLABC]], rows, sem).wait()

    def _scatter(ch, rows, sem):
        dbuf = idx[(ch // SLABC) % 2][1]
        pltpu.async_copy(rows, acc_sh.at[dbuf.at[ch % SLABC]], sem, add=True)

    def _wait_scatter(ch, rows, sem):
        dbuf = idx[(ch // SLABC) % 2][1]
        pltpu.make_async_copy(rows, acc_sh.at[dbuf.at[ch % SLABC]], sem).wait()

    _load_slab(0)

    def zero_body(i, _):
        for j in range(D // 16):
            r0[i, pl.ds(j * 16, 16)] = jnp.zeros((16,), jnp.float32)
        return _

    lax.fori_loop(0, MK, zero_body, None)
    for t in range(RPT // MK):
        pltpu.sync_copy(r0, acc_sh.at[pl.ds(s * RPT + t * MK, MK)])
    _wait_slab(0)
    plsc.subcore_barrier()

    _gather(0, r0, sg0)
    _gather(1, r1, sg1)
    for ch in range(MCHUNK):
        b = ch // SLABC
        cin = ch % SLABC
        if cin == 0 and b + 1 <= NSLAB - 1:
            _load_slab(b + 1)
        if cin == SLABC - 2 and b + 1 <= NSLAB - 1:
            _wait_slab(b + 1)
        rows, sg, ss = ring[ch % NRB]
        _wait_gather(ch, rows, sg)
        _scatter(ch, rows, ss)
        prows, psg, pss = ring[(ch + 2) % NRB]
        if ch >= 2:
            _wait_scatter(ch - 2, prows, pss)
        pch = ch + 2 if ch + 2 < MCHUNK else ch - 2
        _gather(pch, prows, psg)

    for ch in (MCHUNK - 2, MCHUNK - 1):
        rows, sg, ss = ring[ch % NRB]
        _wait_scatter(ch, rows, ss)
    for ch in (MCHUNK - 4, MCHUNK - 3):
        rows, sg, ss = ring[ch % NRB]
        _wait_gather(ch, rows, sg)
    plsc.subcore_barrier()
    pltpu.sync_copy(acc_sh.at[pl.ds(s * RPT, RPT)],
                    out_hbm.at[c, pl.ds(s * RPT, RPT)])


def _row_scale(deg_ref):
    dsum = deg_ref[0] + deg_ref[1] + 1.0
    isq = lax.rsqrt(dsum)
    ones = jnp.ones((1, D), jnp.float32)
    return lax.dot_general(isq, ones, (((0,), (0,)), ((), ())),
                           precision=lax.Precision.HIGHEST,
                           preferred_element_type=jnp.float32)


def _normalize_body(deg_ref, x_ref, out_ref):
    omat = _row_scale(deg_ref)
    out_ref[...] = omat[:N] * x_ref[...]


_normalize = pl.pallas_call(
    _normalize_body,
    out_shape=jax.ShapeDtypeStruct((N, D), jnp.float32),
)


def _combine_body(deg_ref, part_ref, nrm_ref, out_ref):
    omat = _row_scale(deg_ref)
    pooled = part_ref[0][:N] + part_ref[1][:N]
    out_ref[...] = omat[:N] * (pooled + nrm_ref[...])


_combine = pl.pallas_call(
    _combine_body,
    out_shape=jax.ShapeDtypeStruct((N, D), jnp.float32),
)


@jax.jit
def kernel(x, edge_index):
    src = edge_index[0].astype(jnp.int32)
    dst = edge_index[1].astype(jnp.int32)
    npad = EPAD - E
    pad = jnp.arange(npad, dtype=jnp.int32)
    pad_src = pad & 8191
    pad_dst = N + (pad & 127)
    src3 = jnp.concatenate([src, pad_src]).reshape(NW, MCHUNK, MK)
    dst3 = jnp.concatenate([dst, pad_dst]).reshape(NW, MCHUNK, MK)

    degp = _degree_pass(dst3)
    normalized = _normalize(degp, x)
    parts = _message_pass(src3, dst3, normalized)
    return _combine(degp, parts, normalized)
